# baseline (speedup 1.0000x reference)
import jax
import jax.numpy as jnp
from jax.experimental import pallas as pl

_BM = 256


def _ncm_kernel(x_ref, mu_ref, ck_ref, out_ref):
    x = x_ref[...]
    mu = mu_ref[...]
    ck = ck_ref[...]

    xx = jnp.sum(x * x, axis=1, keepdims=True)
    mm = jnp.sum(mu * mu, axis=1)[None, :]
    xm = jax.lax.dot_general(
        x, mu, (((1,), (1,)), ((), ())),
        preferred_element_type=jnp.float32,
    )
    scores = 2.0 * xm - xx - mm

    min_col = jnp.min(scores, axis=1, keepdims=True) - 1.0
    out_ref[...] = jnp.where(ck == 0.0, min_col, scores)


def kernel(X, muK, cK):
    M, d = X.shape
    K = muK.shape[0]
    return pl.pallas_call(
        _ncm_kernel,
        grid=(M // _BM,),
        in_specs=[
            pl.BlockSpec((_BM, d), lambda i: (i, 0)),
            pl.BlockSpec((K, d), lambda i: (0, 0)),
            pl.BlockSpec((1, K), lambda i: (0, 0)),
        ],
        out_specs=pl.BlockSpec((_BM, K), lambda i: (i, 0)),
        out_shape=jax.ShapeDtypeStruct((M, K), jnp.float32),
    )(X, muK, cK.reshape(1, K))


# speedup vs baseline: 1.1263x; 1.1263x over previous
import jax
import jax.numpy as jnp
from jax.experimental import pallas as pl

_BM = 512


def _ncm_kernel(x_ref, mu_ref, ck_ref, out_ref):
    x = x_ref[...]
    mu = mu_ref[...]
    ck = ck_ref[...]

    xx = jnp.sum(x * x, axis=1, keepdims=True)
    mm = jnp.sum(mu * mu, axis=1)[None, :]
    xm = jax.lax.dot_general(
        x, mu, (((1,), (1,)), ((), ())),
        preferred_element_type=jnp.float32,
    )
    scores = 2.0 * xm - xx - mm

    min_col = jnp.min(scores, axis=1, keepdims=True) - 1.0
    out_ref[...] = jnp.where(ck == 0.0, min_col, scores)


def kernel(X, muK, cK):
    M, d = X.shape
    K = muK.shape[0]
    return pl.pallas_call(
        _ncm_kernel,
        grid=(M // _BM,),
        in_specs=[
            pl.BlockSpec((_BM, d), lambda i: (i, 0)),
            pl.BlockSpec((K, d), lambda i: (0, 0)),
            pl.BlockSpec((1, K), lambda i: (0, 0)),
        ],
        out_specs=pl.BlockSpec((_BM, K), lambda i: (i, 0)),
        out_shape=jax.ShapeDtypeStruct((M, K), jnp.float32),
    )(X, muK, cK.reshape(1, K))
